# baseline (speedup 1.0000x reference)
import jax
import jax.numpy as jnp
from jax.experimental import pallas as pl

B = 1
C = (128, 128, 128)
STRIDES = (8, 16, 32)
SCALES = (1.0,)
A = 1
TOPK = 1000
NMS_THR = 0.7
FEAT_HW = ((64, 64), (32, 32), (16, 16))
PKEYS = ('w1', 'b1', 'w2', 'b2', 'wo', 'bo', 'wb', 'bb')


def _conv(x, w, b, pad):
    y = jax.lax.conv_general_dilated(x, w, (1, 1), ((pad, pad), (pad, pad)), dimension_numbers=('NCHW', 'OIHW', 'NCHW'))
    return y + b[None, :, None, None]


def _grid_anchors(H, W, stride):
    ys, xs = jnp.meshgrid(jnp.arange(H, dtype=jnp.float32), jnp.arange(W, dtype=jnp.float32), indexing='ij')
    cx = (xs + 0.5) * stride
    cy = (ys + 0.5) * stride
    out = []
    for s in SCALES:
        w = jnp.full_like(cx, float(stride * s))
        h = jnp.full_like(cy, float(stride * s))
        out.append(jnp.stack([cx, cy, w, h], -1))
    return jnp.stack(out, 0)


def _decode(anchors, deltas):
    cx = anchors[..., 0] + deltas[..., 0] * anchors[..., 2]
    cy = anchors[..., 1] + deltas[..., 1] * anchors[..., 3]
    w = anchors[..., 2] * jnp.exp(deltas[..., 2])
    h = anchors[..., 3] * jnp.exp(deltas[..., 3])
    return jnp.stack([cx - w / 2, cy - h / 2, cx + w / 2, cy + h / 2], -1)


def _nms_keep_sorted(boxes, thr):
    n = boxes.shape[0]
    area = (boxes[:, 2] - boxes[:, 0]) * (boxes[:, 3] - boxes[:, 1])
    lt = jnp.maximum(boxes[:, None, :2], boxes[None, :, :2])
    rb = jnp.minimum(boxes[:, None, 2:], boxes[None, :, 2:])
    wh = jnp.clip(rb - lt, 0.0)
    inter = wh[..., 0] * wh[..., 1]
    iou = inter / (area[:, None] + area[None, :] - inter + 1e-9)
    ar = jnp.arange(n)
    def body(keep, i):
        sup = (iou[i] > thr) & (ar > i) & keep[i]
        return keep & (~sup), None
    keep, _ = jax.lax.scan(body, jnp.ones((n,), dtype=bool), jnp.arange(n))
    return keep


def _batched_nms_sorted(boxes, idxs, thr):
    off = idxs.astype(boxes.dtype) * (jnp.max(boxes) + 1.0)
    return _nms_keep_sorted(boxes + off[:, None], thr)


def _forward(feats, params):
    Bsz = feats[0].shape[0]
    cp, cs, cl = [], [], []
    for lvl in range(3):
        p = params[lvl]
        x = jax.nn.relu(_conv(feats[lvl], p['w1'], p['b1'], 1))
        x = jax.nn.relu(_conv(x, p['w2'], p['b2'], 1))
        obj = jax.nn.sigmoid(_conv(x, p['wo'], p['bo'], 0))
        box = _conv(x, p['wb'], p['bb'], 0)
        Hh, Ww = obj.shape[2], obj.shape[3]
        anchors = _grid_anchors(Hh, Ww, STRIDES[lvl]).reshape(1, A, Hh, Ww, 4)
        box = box.reshape(Bsz, A, 4, Hh, Ww).transpose(0, 1, 3, 4, 2)
        props = _decode(anchors, box).reshape(Bsz, -1, 4)
        scores = obj.reshape(Bsz, -1)
        k = min(scores.shape[1], TOPK * 3)
        sc, idx = jax.lax.top_k(scores, k)
        gp = jnp.take_along_axis(props, idx[:, :, None], axis=1)
        lvls = jnp.full((Bsz, k), lvl, dtype=jnp.int32)
        sc_rows = []
        for b in range(Bsz):
            keep = _batched_nms_sorted(jax.lax.stop_gradient(gp[b]), lvls[b], NMS_THR)
            rank = jnp.cumsum(keep.astype(jnp.int32)) - 1
            valid = keep & (rank < TOPK)
            sc_rows.append(jnp.where(valid, sc[b], -jnp.inf))
        cp.append(gp)
        cs.append(jnp.stack(sc_rows, 0))
        cl.append(lvls)
    props_cat = jnp.concatenate(cp, 1)
    scores_cat = jnp.concatenate(cs, 1)
    lvls_cat = jnp.concatenate(cl, 1)
    Kf = min(TOPK, scores_cat.shape[1])
    sc, idx = jax.lax.top_k(scores_cat, Kf)
    props_final = jnp.take_along_axis(props_cat, idx[:, :, None], axis=1)
    lvls_final = jnp.take_along_axis(lvls_cat, idx, axis=1)
    return props_final, sc, lvls_final


def kernel(feat0, feat1, feat2, l0_w1, l0_b1, l0_w2, l0_b2, l0_wo, l0_bo, l0_wb, l0_bb, l1_w1, l1_b1, l1_w2, l1_b2, l1_wo, l1_bo, l1_wb, l1_bb, l2_w1, l2_b1, l2_w2, l2_b2, l2_wo, l2_bo, l2_wb, l2_bb):
    feats = [feat0, feat1, feat2]
    kw = dict(l0_w1=l0_w1, l0_b1=l0_b1, l0_w2=l0_w2, l0_b2=l0_b2,
              l0_wo=l0_wo, l0_bo=l0_bo, l0_wb=l0_wb, l0_bb=l0_bb,
              l1_w1=l1_w1, l1_b1=l1_b1, l1_w2=l1_w2, l1_b2=l1_b2,
              l1_wo=l1_wo, l1_bo=l1_bo, l1_wb=l1_wb, l1_bb=l1_bb,
              l2_w1=l2_w1, l2_b1=l2_b1, l2_w2=l2_w2, l2_b2=l2_b2,
              l2_wo=l2_wo, l2_bo=l2_bo, l2_wb=l2_wb, l2_bb=l2_bb)
    params = [{k: kw['l%d_%s' % (i, k)] for k in PKEYS} for i in range(3)]
    return _forward(feats, params)


# speedup vs baseline: 84.0072x; 84.0072x over previous
import functools

import jax
import jax.numpy as jnp
from jax.experimental import pallas as pl
from jax.experimental.pallas import tpu as pltpu

B = 1
C = (128, 128, 128)
STRIDES = (8, 16, 32)
SCALES = (1.0,)
A = 1
TOPK = 1000
NMS_THR = 0.7
FEAT_HW = ((64, 64), (32, 32), (16, 16))
PKEYS = ('w1', 'b1', 'w2', 'b2', 'wo', 'bo', 'wb', 'bb')

NEG = float('-inf')


def _conv(x, w, b, pad):
    y = jax.lax.conv_general_dilated(x, w, (1, 1), ((pad, pad), (pad, pad)), dimension_numbers=('NCHW', 'OIHW', 'NCHW'))
    return y + b[None, :, None, None]


def _grid_anchors(H, W, stride):
    ys, xs = jnp.meshgrid(jnp.arange(H, dtype=jnp.float32), jnp.arange(W, dtype=jnp.float32), indexing='ij')
    cx = (xs + 0.5) * stride
    cy = (ys + 0.5) * stride
    out = []
    for s in SCALES:
        w = jnp.full_like(cx, float(stride * s))
        h = jnp.full_like(cy, float(stride * s))
        out.append(jnp.stack([cx, cy, w, h], -1))
    return jnp.stack(out, 0)


def _decode(anchors, deltas):
    cx = anchors[..., 0] + deltas[..., 0] * anchors[..., 2]
    cy = anchors[..., 1] + deltas[..., 1] * anchors[..., 3]
    w = anchors[..., 2] * jnp.exp(deltas[..., 2])
    h = anchors[..., 3] * jnp.exp(deltas[..., 3])
    return jnp.stack([cx - w / 2, cy - h / 2, cx + w / 2, cy + h / 2], -1)


def _rank_body(scol_ref, srow_ref, out_ref, *, Sj, N):
    j = pl.program_id(0)
    sj = scol_ref[...]
    srow = srow_ref[...]
    jidx = jax.lax.broadcasted_iota(jnp.int32, (Sj, 1), 0) + j * Sj
    iidx = jax.lax.broadcasted_iota(jnp.int32, (1, N), 1)
    cmp = (sj > srow) | ((sj == srow) & (jidx < iidx))
    part = jnp.sum(cmp.astype(jnp.float32), axis=0, keepdims=True)

    @pl.when(j == 0)
    def _():
        out_ref[...] = jnp.zeros_like(out_ref)

    out_ref[...] += part


def _rank_call(scores_row, Sj=256):
    N = scores_row.shape[1]
    scol = scores_row.reshape(N, 1)
    return pl.pallas_call(
        functools.partial(_rank_body, Sj=Sj, N=N),
        grid=(N // Sj,),
        in_specs=[
            pl.BlockSpec((Sj, 1), lambda j: (j, 0)),
            pl.BlockSpec((1, N), lambda j: (0, 0)),
        ],
        out_specs=pl.BlockSpec((1, N), lambda j: (0, 0)),
        out_shape=jax.ShapeDtypeStruct((1, N), jnp.float32),
    )(scol, scores_row)


def _gather_body(rank_ref, dataT_ref, out_ref, *, Rb, N, NC):
    r = pl.program_id(0)
    rank = rank_ref[...]
    ridx = (jax.lax.broadcasted_iota(jnp.int32, (Rb, 1), 0) + r * Rb).astype(jnp.float32)
    P = rank == ridx
    for c in range(NC):
        drow = dataT_ref[c:c + 1, :]
        sel = jnp.where(P, drow, NEG)
        out_ref[:, c:c + 1] = jnp.max(sel, axis=1, keepdims=True)


def _gather_call(rank_row, dataT, K, Rb=256):
    NC, N = dataT.shape
    return pl.pallas_call(
        functools.partial(_gather_body, Rb=Rb, N=N, NC=NC),
        grid=(K // Rb,),
        in_specs=[
            pl.BlockSpec((1, N), lambda r: (0, 0)),
            pl.BlockSpec((NC, N), lambda r: (0, 0)),
        ],
        out_specs=pl.BlockSpec((Rb, NC), lambda r: (r, 0)),
        out_shape=jax.ShapeDtypeStruct((K, NC), jnp.float32),
    )(rank_row, dataT)


def _nms_body(data_ref, dataT_ref, out_ref, keep_ref, sup_ref, *, K, k_real, S, lvl, thr, topk):
    nb = K // S
    f32 = jnp.float32

    boxes = data_ref[:, 0:4]
    rmask = jax.lax.broadcasted_iota(jnp.int32, (K, 1), 0) < k_real
    bmax = jnp.max(jnp.where(rmask, boxes, NEG))
    off = jnp.float32(lvl) * (bmax + 1.0)

    def cols(lo, n):
        x0 = data_ref[lo:lo + n, 0:1] + off
        y0 = data_ref[lo:lo + n, 1:2] + off
        x1 = data_ref[lo:lo + n, 2:3] + off
        y1 = data_ref[lo:lo + n, 3:4] + off
        ar = (x1 - x0) * (y1 - y0)
        return x0, y0, x1, y1, ar

    def rows(lo, n):
        x0 = dataT_ref[0:1, lo:lo + n] + off
        y0 = dataT_ref[1:2, lo:lo + n] + off
        x1 = dataT_ref[2:3, lo:lo + n] + off
        y1 = dataT_ref[3:4, lo:lo + n] + off
        ar = (x1 - x0) * (y1 - y0)
        return x0, y0, x1, y1, ar

    def iou_gt(ca, ra):
        cx0, cy0, cx1, cy1, car = ca
        rx0, ry0, rx1, ry1, rar = ra
        ltx = jnp.maximum(cx0, rx0)
        lty = jnp.maximum(cy0, ry0)
        rbx = jnp.minimum(cx1, rx1)
        rby = jnp.minimum(cy1, ry1)
        whx = jnp.maximum(rbx - ltx, 0.0)
        why = jnp.maximum(rby - lty, 0.0)
        inter = whx * why
        iou = inter / (car + rar - inter + 1e-9)
        return (iou > thr).astype(f32)

    sup_ref[...] = jnp.zeros((1, K), f32)

    for b in range(nb):
        lo = b * S
        ca = cols(lo, S)
        ra = rows(lo, S)
        M = iou_gt(ca, ra)
        upper = (jax.lax.broadcasted_iota(jnp.int32, (S, 1), 0)
                 < jax.lax.broadcasted_iota(jnp.int32, (1, S), 1)).astype(f32)
        M = M * upper
        pre = (sup_ref[0:1, lo:lo + S] == 0.0).astype(f32)

        def cond(c):
            return c[1]

        def body(c):
            kb, _ = c
            hit = jnp.dot(kb, M, preferred_element_type=f32)
            kb2 = pre * (hit == 0.0).astype(f32)
            return kb2, jnp.any(kb2 != kb)

        kb, _ = jax.lax.while_loop(cond, body, (pre, True))
        keep_ref[0:1, lo:lo + S] = kb

        rest = K - (lo + S)
        if rest > 0:
            for b2 in range(b + 1, nb):
                lo2 = b2 * S
                M12 = iou_gt(ca, rows(lo2, S))
                sup_ref[0:1, lo2:lo2 + S] += jnp.dot(kb, M12, preferred_element_type=f32)

    keep = keep_ref[...]
    csum = jnp.zeros((1, 0), f32)
    running = jnp.zeros((), f32)
    U = (jax.lax.broadcasted_iota(jnp.int32, (S, 1), 0)
         <= jax.lax.broadcasted_iota(jnp.int32, (1, S), 1)).astype(f32)
    parts = []
    for b in range(nb):
        lo = b * S
        kb = keep[0:1, lo:lo + S]
        local = jnp.dot(kb, U, preferred_element_type=f32) + running
        parts.append(local)
        running = running + jnp.sum(kb)
    csum = jnp.concatenate(parts, axis=1)

    score = dataT_ref[4:5, :]
    idx = jax.lax.broadcasted_iota(jnp.int32, (1, K), 1)
    valid = (keep != 0.0) & (csum <= jnp.float32(topk)) & (idx < k_real)
    out_ref[...] = jnp.where(valid, score, NEG)


def _nms_call(data, dataT, k_real, S, lvl):
    K = data.shape[0]
    return pl.pallas_call(
        functools.partial(_nms_body, K=K, k_real=k_real, S=S, lvl=lvl,
                          thr=NMS_THR, topk=TOPK),
        scratch_shapes=[
            pltpu.VMEM((1, K), jnp.float32),
            pltpu.VMEM((1, K), jnp.float32),
        ],
        out_shape=jax.ShapeDtypeStruct((1, K), jnp.float32),
    )(data, dataT)


def kernel(feat0, feat1, feat2, l0_w1, l0_b1, l0_w2, l0_b2, l0_wo, l0_bo, l0_wb, l0_bb, l1_w1, l1_b1, l1_w2, l1_b2, l1_wo, l1_bo, l1_wb, l1_bb, l2_w1, l2_b1, l2_w2, l2_b2, l2_wo, l2_bo, l2_wb, l2_bb):
    feats = [feat0, feat1, feat2]
    kw = dict(l0_w1=l0_w1, l0_b1=l0_b1, l0_w2=l0_w2, l0_b2=l0_b2,
              l0_wo=l0_wo, l0_bo=l0_bo, l0_wb=l0_wb, l0_bb=l0_bb,
              l1_w1=l1_w1, l1_b1=l1_b1, l1_w2=l1_w2, l1_b2=l1_b2,
              l1_wo=l1_wo, l1_bo=l1_bo, l1_wb=l1_wb, l1_bb=l1_bb,
              l2_w1=l2_w1, l2_b1=l2_b1, l2_w2=l2_w2, l2_b2=l2_b2,
              l2_wo=l2_wo, l2_bo=l2_bo, l2_wb=l2_wb, l2_bb=l2_bb)
    params = [{k: kw['l%d_%s' % (i, k)] for k in PKEYS} for i in range(3)]

    KPAD = (3072, 1024, 256)
    KREAL = (3000, 1024, 256)
    SBLK = (512, 512, 256)

    masked, sorted_data = [], []
    for lvl in range(3):
        p = params[lvl]
        x = jax.nn.relu(_conv(feats[lvl], p['w1'], p['b1'], 1))
        x = jax.nn.relu(_conv(x, p['w2'], p['b2'], 1))
        obj = jax.nn.sigmoid(_conv(x, p['wo'], p['bo'], 0))
        box = _conv(x, p['wb'], p['bb'], 0)
        Hh, Ww = obj.shape[2], obj.shape[3]
        N = Hh * Ww
        anchors = _grid_anchors(Hh, Ww, STRIDES[lvl]).reshape(1, A, Hh, Ww, 4)
        box = box.reshape(B, A, 4, Hh, Ww).transpose(0, 1, 3, 4, 2)
        props = _decode(anchors, box).reshape(B, -1, 4)
        scores_row = obj.reshape(1, N)

        data8 = jnp.concatenate([
            props[0],
            scores_row.reshape(N, 1),
            jnp.full((N, 1), float(lvl), jnp.float32),
            jnp.zeros((N, 2), jnp.float32),
        ], axis=1)

        rank = _rank_call(scores_row)
        srt = _gather_call(rank, data8.T, KPAD[lvl])
        m = _nms_call(srt, srt.T, KREAL[lvl], SBLK[lvl], lvl)
        masked.append(m[:, :KREAL[lvl]])
        sorted_data.append(srt[:KREAL[lvl]])

    M = sum(KREAL)
    MPAD = 4352
    pad = MPAD - M
    scat = jnp.concatenate(masked + [jnp.full((1, pad), NEG, jnp.float32)], axis=1)
    dcat = jnp.concatenate(sorted_data + [jnp.zeros((pad, 8), jnp.float32)], axis=0)

    frank = _rank_call(scat, Sj=256)
    out8 = _gather_call(frank, dcat.T, 1024)

    props_final = out8[:TOPK, 0:4].reshape(1, TOPK, 4)
    sc = out8[:TOPK, 4].reshape(1, TOPK)
    lvls = out8[:TOPK, 5].astype(jnp.int32).reshape(1, TOPK)
    return props_final, sc, lvls


# speedup vs baseline: 124.3376x; 1.4801x over previous
import functools

import jax
import jax.numpy as jnp
from jax.experimental import pallas as pl
from jax.experimental.pallas import tpu as pltpu
from jax.experimental.pallas import tpu_sc as plsc

B = 1
C = (128, 128, 128)
STRIDES = (8, 16, 32)
SCALES = (1.0,)
A = 1
TOPK = 1000
NMS_THR = 0.7
FEAT_HW = ((64, 64), (32, 32), (16, 16))
PKEYS = ('w1', 'b1', 'w2', 'b2', 'wo', 'bo', 'wb', 'bb')

NEG = float('-inf')


def _conv(x, w, b, pad):
    y = jax.lax.conv_general_dilated(x, w, (1, 1), ((pad, pad), (pad, pad)), dimension_numbers=('NCHW', 'OIHW', 'NCHW'))
    return y + b[None, :, None, None]


def _grid_anchors(H, W, stride):
    ys, xs = jnp.meshgrid(jnp.arange(H, dtype=jnp.float32), jnp.arange(W, dtype=jnp.float32), indexing='ij')
    cx = (xs + 0.5) * stride
    cy = (ys + 0.5) * stride
    out = []
    for s in SCALES:
        w = jnp.full_like(cx, float(stride * s))
        h = jnp.full_like(cy, float(stride * s))
        out.append(jnp.stack([cx, cy, w, h], -1))
    return jnp.stack(out, 0)


def _decode(anchors, deltas):
    cx = anchors[..., 0] + deltas[..., 0] * anchors[..., 2]
    cy = anchors[..., 1] + deltas[..., 1] * anchors[..., 3]
    w = anchors[..., 2] * jnp.exp(deltas[..., 2])
    h = anchors[..., 3] * jnp.exp(deltas[..., 3])
    return jnp.stack([cx - w / 2, cy - h / 2, cx + w / 2, cy + h / 2], -1)


def _rank_body(scol_ref, srow_ref, out_ref, *, Sj, N):
    j = pl.program_id(0)
    sj = scol_ref[...]
    srow = srow_ref[...]
    jidx = jax.lax.broadcasted_iota(jnp.int32, (Sj, 1), 0) + j * Sj
    iidx = jax.lax.broadcasted_iota(jnp.int32, (1, N), 1)
    cmp = (sj > srow) | ((sj == srow) & (jidx < iidx))
    part = jnp.sum(cmp.astype(jnp.float32), axis=0, keepdims=True)

    @pl.when(j == 0)
    def _():
        out_ref[...] = jnp.zeros_like(out_ref)

    out_ref[...] += part


def _rank_call(scores_row, Sj=256):
    N = scores_row.shape[1]
    scol = scores_row.reshape(N, 1)
    return pl.pallas_call(
        functools.partial(_rank_body, Sj=Sj, N=N),
        grid=(N // Sj,),
        in_specs=[
            pl.BlockSpec((Sj, 1), lambda j: (j, 0)),
            pl.BlockSpec((1, N), lambda j: (0, 0)),
        ],
        out_specs=pl.BlockSpec((1, N), lambda j: (0, 0)),
        out_shape=jax.ShapeDtypeStruct((1, N), jnp.float32),
    )(scol, scores_row)


def _gather_body(rank_ref, dataT_ref, out_ref, *, Rb, N, NC):
    r = pl.program_id(0)
    rank = rank_ref[...]
    ridx = (jax.lax.broadcasted_iota(jnp.int32, (Rb, 1), 0) + r * Rb).astype(jnp.float32)
    P = rank == ridx
    for c in range(NC):
        drow = dataT_ref[c:c + 1, :]
        sel = jnp.where(P, drow, NEG)
        out_ref[:, c:c + 1] = jnp.max(sel, axis=1, keepdims=True)


def _gather_call(rank_row, dataT, K, Rb=256):
    NC, N = dataT.shape
    return pl.pallas_call(
        functools.partial(_gather_body, Rb=Rb, N=N, NC=NC),
        grid=(K // Rb,),
        in_specs=[
            pl.BlockSpec((1, N), lambda r: (0, 0)),
            pl.BlockSpec((NC, N), lambda r: (0, 0)),
        ],
        out_specs=pl.BlockSpec((Rb, NC), lambda r: (r, 0)),
        out_shape=jax.ShapeDtypeStruct((K, NC), jnp.float32),
    )(rank_row, dataT)


def _sc_permute(data128, rank_i32):
    N = data128.shape[0]
    info = plsc.get_sparse_core_info()
    NW = info.num_cores * info.num_subcores
    chunk = N // NW
    mesh = plsc.VectorSubcoreMesh(core_axis_name="c", subcore_axis_name="s")

    @functools.partial(
        pl.kernel, mesh=mesh,
        out_type=jax.ShapeDtypeStruct((N, 128), jnp.float32),
        scratch_types=[
            pltpu.VMEM((chunk,), jnp.int32),
            pltpu.VMEM((chunk, 128), jnp.float32),
            pltpu.SemaphoreType.DMA,
        ],
    )
    def k(data_hbm, idx_hbm, out_hbm, idx_v, rows_v, sem):
        wid = jax.lax.axis_index("s") * info.num_cores + jax.lax.axis_index("c")
        base = wid * chunk
        pltpu.sync_copy(idx_hbm.at[pl.ds(base, chunk)], idx_v)
        pltpu.sync_copy(data_hbm.at[pl.ds(base, chunk)], rows_v)
        pltpu.async_copy(rows_v, out_hbm.at[idx_v], sem).wait()

    return k(data128, rank_i32)


def _nms_body(data_ref, dataT_ref, out_ref, keep_ref, sup_ref, *, K, k_real, S, lvl, thr, topk):
    nb = K // S
    f32 = jnp.float32

    boxes = data_ref[:, 0:4]
    rmask = jax.lax.broadcasted_iota(jnp.int32, (K, 1), 0) < k_real
    bmax = jnp.max(jnp.where(rmask, boxes, NEG))
    off = jnp.float32(lvl) * (bmax + 1.0)

    def cols(lo, n):
        x0 = data_ref[lo:lo + n, 0:1] + off
        y0 = data_ref[lo:lo + n, 1:2] + off
        x1 = data_ref[lo:lo + n, 2:3] + off
        y1 = data_ref[lo:lo + n, 3:4] + off
        ar = (x1 - x0) * (y1 - y0)
        return x0, y0, x1, y1, ar

    def rows(lo, n):
        x0 = dataT_ref[0:1, lo:lo + n] + off
        y0 = dataT_ref[1:2, lo:lo + n] + off
        x1 = dataT_ref[2:3, lo:lo + n] + off
        y1 = dataT_ref[3:4, lo:lo + n] + off
        ar = (x1 - x0) * (y1 - y0)
        return x0, y0, x1, y1, ar

    def iou_gt(ca, ra):
        cx0, cy0, cx1, cy1, car = ca
        rx0, ry0, rx1, ry1, rar = ra
        ltx = jnp.maximum(cx0, rx0)
        lty = jnp.maximum(cy0, ry0)
        rbx = jnp.minimum(cx1, rx1)
        rby = jnp.minimum(cy1, ry1)
        whx = jnp.maximum(rbx - ltx, 0.0)
        why = jnp.maximum(rby - lty, 0.0)
        inter = whx * why
        iou = inter / (car + rar - inter + 1e-9)
        return (iou > thr).astype(f32)

    sup_ref[...] = jnp.zeros((1, K), f32)

    for b in range(nb):
        lo = b * S
        ca = cols(lo, S)
        ra = rows(lo, S)
        M = iou_gt(ca, ra)
        upper = (jax.lax.broadcasted_iota(jnp.int32, (S, 1), 0)
                 < jax.lax.broadcasted_iota(jnp.int32, (1, S), 1)).astype(f32)
        M = M * upper
        pre = (sup_ref[0:1, lo:lo + S] == 0.0).astype(f32)

        def cond(c):
            return c[1]

        def body(c):
            kb, _ = c
            hit = jnp.dot(kb, M, preferred_element_type=f32)
            kb2 = pre * (hit == 0.0).astype(f32)
            return kb2, jnp.any(kb2 != kb)

        kb, _ = jax.lax.while_loop(cond, body, (pre, True))
        keep_ref[0:1, lo:lo + S] = kb

        rest = K - (lo + S)
        if rest > 0:
            for b2 in range(b + 1, nb):
                lo2 = b2 * S
                M12 = iou_gt(ca, rows(lo2, S))
                sup_ref[0:1, lo2:lo2 + S] += jnp.dot(kb, M12, preferred_element_type=f32)

    keep = keep_ref[...]
    csum = jnp.zeros((1, 0), f32)
    running = jnp.zeros((), f32)
    U = (jax.lax.broadcasted_iota(jnp.int32, (S, 1), 0)
         <= jax.lax.broadcasted_iota(jnp.int32, (1, S), 1)).astype(f32)
    parts = []
    for b in range(nb):
        lo = b * S
        kb = keep[0:1, lo:lo + S]
        local = jnp.dot(kb, U, preferred_element_type=f32) + running
        parts.append(local)
        running = running + jnp.sum(kb)
    csum = jnp.concatenate(parts, axis=1)

    score = dataT_ref[4:5, :]
    idx = jax.lax.broadcasted_iota(jnp.int32, (1, K), 1)
    valid = (keep != 0.0) & (csum <= jnp.float32(topk)) & (idx < k_real)
    out_ref[...] = jnp.where(valid, score, NEG)


def _nms_call(data, dataT, k_real, S, lvl):
    K = data.shape[0]
    return pl.pallas_call(
        functools.partial(_nms_body, K=K, k_real=k_real, S=S, lvl=lvl,
                          thr=NMS_THR, topk=TOPK),
        scratch_shapes=[
            pltpu.VMEM((1, K), jnp.float32),
            pltpu.VMEM((1, K), jnp.float32),
        ],
        out_shape=jax.ShapeDtypeStruct((1, K), jnp.float32),
    )(data, dataT)


def kernel(feat0, feat1, feat2, l0_w1, l0_b1, l0_w2, l0_b2, l0_wo, l0_bo, l0_wb, l0_bb, l1_w1, l1_b1, l1_w2, l1_b2, l1_wo, l1_bo, l1_wb, l1_bb, l2_w1, l2_b1, l2_w2, l2_b2, l2_wo, l2_bo, l2_wb, l2_bb):
    feats = [feat0, feat1, feat2]
    kw = dict(l0_w1=l0_w1, l0_b1=l0_b1, l0_w2=l0_w2, l0_b2=l0_b2,
              l0_wo=l0_wo, l0_bo=l0_bo, l0_wb=l0_wb, l0_bb=l0_bb,
              l1_w1=l1_w1, l1_b1=l1_b1, l1_w2=l1_w2, l1_b2=l1_b2,
              l1_wo=l1_wo, l1_bo=l1_bo, l1_wb=l1_wb, l1_bb=l1_bb,
              l2_w1=l2_w1, l2_b1=l2_b1, l2_w2=l2_w2, l2_b2=l2_b2,
              l2_wo=l2_wo, l2_bo=l2_bo, l2_wb=l2_wb, l2_bb=l2_bb)
    params = [{k: kw['l%d_%s' % (i, k)] for k in PKEYS} for i in range(3)]

    KPAD = (3072, 1024, 256)
    SBLK = (512, 512, 256)

    masked, sorted_data = [], []
    for lvl in range(3):
        p = params[lvl]
        x = jax.nn.relu(_conv(feats[lvl], p['w1'], p['b1'], 1))
        x = jax.nn.relu(_conv(x, p['w2'], p['b2'], 1))
        obj = jax.nn.sigmoid(_conv(x, p['wo'], p['bo'], 0))
        box = _conv(x, p['wb'], p['bb'], 0)
        Hh, Ww = obj.shape[2], obj.shape[3]
        N = Hh * Ww
        anchors = _grid_anchors(Hh, Ww, STRIDES[lvl]).reshape(1, A, Hh, Ww, 4)
        box = box.reshape(B, A, 4, Hh, Ww).transpose(0, 1, 3, 4, 2)
        props = _decode(anchors, box).reshape(B, -1, 4)
        scores = obj.reshape(B, N)

        k = min(N, TOPK * 3)
        sc, idx = jax.lax.top_k(scores, k)
        gp = jnp.take_along_axis(props, idx[:, :, None], axis=1)

        data8 = jnp.concatenate([
            gp[0],
            sc[0].reshape(k, 1),
            jnp.full((k, 1), float(lvl), jnp.float32),
            jnp.zeros((k, 2), jnp.float32),
        ], axis=1)
        if KPAD[lvl] > k:
            data8p = jnp.concatenate(
                [data8, jnp.zeros((KPAD[lvl] - k, 8), jnp.float32)], axis=0)
        else:
            data8p = data8

        m = _nms_call(data8p, data8p.T, k, SBLK[lvl], lvl)
        masked.append(m[:, :k])
        sorted_data.append(data8)

    M = sum(d.shape[0] for d in sorted_data)
    MPAD = 4352
    pad = MPAD - M
    scat = jnp.concatenate(masked + [jnp.full((1, pad), NEG, jnp.float32)], axis=1)
    dcat = jnp.concatenate(sorted_data + [jnp.zeros((pad, 8), jnp.float32)], axis=0)
    dcat128 = jnp.concatenate([dcat, jnp.zeros((MPAD, 120), jnp.float32)], axis=1)

    frank = _rank_call(scat, Sj=256)
    out128 = _sc_permute(dcat128, frank.reshape(MPAD).astype(jnp.int32))
    out8 = out128[:1024, :8]

    props_final = out8[:TOPK, 0:4].reshape(1, TOPK, 4)
    sc = out8[:TOPK, 4].reshape(1, TOPK)
    lvls = out8[:TOPK, 5].astype(jnp.int32).reshape(1, TOPK)
    return props_final, sc, lvls
